# baseline (speedup 1.0000x reference)
import functools

import jax
import jax.numpy as jnp
from jax import lax
from jax.experimental import pallas as pl
from jax.experimental.pallas import tpu as pltpu
from jax.experimental.pallas import tpu_sc as plsc

F32 = jnp.float32
BF16 = jnp.bfloat16

B = 8192
D_IN = 768
H1 = 2048
H2 = 1024
D = 256
K = 8192
RB = 512
CB = 2048
EPS = 1e-5


def _ln_elu(a, g, b):
    mu = jnp.mean(a, axis=-1, keepdims=True)
    var = jnp.mean((a - mu) ** 2, axis=-1, keepdims=True)
    y = (a - mu) / jnp.sqrt(var + EPS) * g + b
    return jnp.where(y > 0, y, jnp.exp(y) - 1.0)


def _enc_body(x_ref, we1, be1, g1, bn1, we2, be2, g2, bn2, we3, be3,
              cbt, cbsq, e_ref, idx_ref):
    h = jnp.dot(x_ref[...].astype(BF16), we1[...],
                preferred_element_type=F32) + be1[...]
    h = _ln_elu(h, g1[...], bn1[...])
    h = jnp.dot(h.astype(BF16), we2[...],
                preferred_element_type=F32) + be2[...]
    h = _ln_elu(h, g2[...], bn2[...])
    e = jnp.dot(h.astype(BF16), we3[...],
                preferred_element_type=F32) + be3[...]
    e_ref[...] = e
    ebf2 = (e * -2.0).astype(BF16)
    esq = jnp.sum(e * e, axis=-1, keepdims=True)
    bv = jnp.full((RB, 128), jnp.inf, F32)
    bi = jnp.zeros((RB, 128), jnp.int32)
    for j in range(K // CB):
        d2 = jnp.dot(ebf2, cbt[:, j * CB:(j + 1) * CB],
                     preferred_element_type=F32)
        t = (esq + d2) + cbsq[:, j * CB:(j + 1) * CB]
        for k in range(CB // 128):
            step = j * (CB // 128) + k
            tk = lax.slice(t, (0, k * 128), (RB, (k + 1) * 128))
            pred = tk < bv
            bv = jnp.where(pred, tk, bv)
            bi = jnp.where(pred, jnp.int32(step), bi)
    lanes = lax.broadcasted_iota(jnp.int32, (RB, 128), 1)
    gidx = bi * 128 + lanes
    m = jnp.min(bv, axis=-1, keepdims=True)
    cand = jnp.where(bv == m, gidx, jnp.int32(2**31 - 1))
    idx_ref[...] = jnp.min(cand, axis=-1, keepdims=True)


def _dec_body(e_ref, q_ref, wd1, bd1, gd1, bdn1, wd2, bd2, gd2, bdn2,
              wd3, bd3, xhat_ref, loss_ref):
    hb = RB // 2
    parts = []
    for s in range(2):
        sl = pl.ds(s * hb, hb)
        e = e_ref[sl, :]
        q = q_ref[sl, :]
        diff = e - q
        parts.append(jnp.sum(jnp.sum(diff * diff, axis=-1, keepdims=True),
                             axis=0, keepdims=True))
        q_st = e + (q - e)
        h = jnp.dot(q_st.astype(BF16), wd1[...],
                    preferred_element_type=F32) + bd1[...]
        h = _ln_elu(h, gd1[...], bdn1[...])
        h = jnp.dot(h.astype(BF16), wd2[...],
                    preferred_element_type=F32) + bd2[...]
        h = _ln_elu(h, gd2[...], bdn2[...])
        xhat_ref[sl, :] = (jnp.dot(h.astype(BF16), wd3[...],
                                   preferred_element_type=F32) + bd3[...])
    part = parts[0] + parts[1]

    @pl.when(pl.program_id(0) == 0)
    def _():
        loss_ref[...] = part

    @pl.when(pl.program_id(0) != 0)
    def _():
        loss_ref[...] += part


def _full(shape):
    return pl.BlockSpec(shape, lambda i: (0,) * len(shape))


def _encode(x_bf, we1t, be1, g1, bn1, we2t, be2, g2, bn2, we3t, be3,
            cbt, cbsq):
    nb = x_bf.shape[0]
    return pl.pallas_call(
        _enc_body,
        grid=(nb // RB,),
        in_specs=[
            pl.BlockSpec((RB, D_IN), lambda i: (i, 0)),
            _full((D_IN, H1)), _full((1, H1)), _full((1, H1)), _full((1, H1)),
            _full((H1, H2)), _full((1, H2)), _full((1, H2)), _full((1, H2)),
            _full((H2, D)), _full((1, D)),
            _full((D, K)), _full((1, K)),
        ],
        out_specs=[
            pl.BlockSpec((RB, D), lambda i: (i, 0)),
            pl.BlockSpec((RB, 1), lambda i: (i, 0)),
        ],
        out_shape=[
            jax.ShapeDtypeStruct((nb, D), F32),
            jax.ShapeDtypeStruct((nb, 1), jnp.int32),
        ],
    )(x_bf, we1t, be1, g1, bn1, we2t, be2, g2, bn2, we3t, be3, cbt, cbsq)


def _decode(e, q, wd1t, bd1, gd1, bdn1, wd2t, bd2, gd2, bdn2, wd3t, bd3):
    nb = e.shape[0]
    return pl.pallas_call(
        _dec_body,
        grid=(nb // RB,),
        in_specs=[
            pl.BlockSpec((RB, D), lambda i: (i, 0)),
            pl.BlockSpec((RB, D), lambda i: (i, 0)),
            _full((D, H2)), _full((1, H2)), _full((1, H2)), _full((1, H2)),
            _full((H2, H1)), _full((1, H1)), _full((1, H1)), _full((1, H1)),
            _full((H1, D_IN)), _full((1, D_IN)),
        ],
        out_specs=[
            pl.BlockSpec((RB, D_IN), lambda i: (i, 0)),
            pl.BlockSpec((1, 1), lambda i: (0, 0)),
        ],
        out_shape=[
            jax.ShapeDtypeStruct((nb, D_IN), F32),
            jax.ShapeDtypeStruct((1, 1), F32),
        ],
    )(e, q, wd1t, bd1, gd1, bdn1, wd2t, bd2, gd2, bdn2, wd3t, bd3)


def _gather(codebook, idx):
    nb = idx.shape[0]
    info = plsc.get_sparse_core_info()
    nc, ns = info.num_cores, info.num_subcores
    nw = nc * ns
    bpw = nb // nw
    mesh = plsc.VectorSubcoreMesh(core_axis_name="c", subcore_axis_name="s")

    @functools.partial(
        pl.kernel, mesh=mesh,
        out_type=jax.ShapeDtypeStruct((nb, D), F32),
        scratch_types=[
            pltpu.VMEM((bpw,), jnp.int32),
            pltpu.VMEM((bpw, D), F32),
            pltpu.SemaphoreType.DMA,
            pltpu.SemaphoreType.DMA,
        ],
    )
    def k(table_hbm, idx_hbm, out_hbm, idx_v, rows_v, gsem, ssem):
        wid = lax.axis_index("s") * nc + lax.axis_index("c")
        base = wid * bpw
        nch = 4
        ch = bpw // nch
        pltpu.sync_copy(idx_hbm.at[pl.ds(base, bpw)], idx_v)
        gets = [pltpu.async_copy(table_hbm.at[idx_v.at[pl.ds(c * ch, ch)]],
                                 rows_v.at[pl.ds(c * ch, ch)], gsem)
                for c in range(nch)]
        puts = []
        for c in range(nch):
            gets[c].wait()
            puts.append(pltpu.async_copy(
                rows_v.at[pl.ds(c * ch, ch)],
                out_hbm.at[pl.ds(base + c * ch, ch)], ssem))
        for p in puts:
            p.wait()

    return k(codebook, idx)


def kernel(x, We1, be1, g1, bn1, We2, be2, g2, bn2, We3, be3, codebook,
           Wd1, bd1, gd1, bdn1, Wd2, bd2, gd2, bdn2, Wd3, bd3):
    we1t = We1.T.astype(BF16)
    we2t = We2.T.astype(BF16)
    we3t = We3.T.astype(BF16)
    wd1t = Wd1.T.astype(BF16)
    wd2t = Wd2.T.astype(BF16)
    wd3t = Wd3.T.astype(BF16)
    cbt = codebook.T.astype(BF16)
    cbsq = jnp.sum(codebook ** 2, axis=1).reshape(1, K)
    r = lambda v: v.reshape(1, -1)

    e, idx2 = _encode(x, we1t, r(be1), r(g1), r(bn1), we2t, r(be2),
                      r(g2), r(bn2), we3t, r(be3), cbt, cbsq)
    idx = idx2.reshape(B)
    q = _gather(codebook, idx)
    xhat, loss11 = _decode(e, q, wd1t, r(bd1), r(gd1), r(bdn1), wd2t,
                           r(bd2), r(gd2), r(bdn2), wd3t, r(bd3))
    m = loss11.reshape(()) / (B * D)
    vq_loss = m + 0.25 * m
    return (xhat, vq_loss, idx)


# speedup vs baseline: 1.0287x; 1.0287x over previous
import functools

import jax
import jax.numpy as jnp
from jax import lax
from jax.experimental import pallas as pl
from jax.experimental.pallas import tpu as pltpu
from jax.experimental.pallas import tpu_sc as plsc

F32 = jnp.float32
BF16 = jnp.bfloat16

B = 8192
D_IN = 768
H1 = 2048
H2 = 1024
D = 256
K = 8192
RB = 1024
CB = 2048
EPS = 1e-5


def _ln_elu(a, g, b):
    mu = jnp.mean(a, axis=-1, keepdims=True)
    var = jnp.mean((a - mu) ** 2, axis=-1, keepdims=True)
    y = (a - mu) / jnp.sqrt(var + EPS) * g + b
    return jnp.where(y > 0, y, jnp.exp(y) - 1.0)


def _enc_body(x_ref, we1, be1, g1, bn1, we2, be2, g2, bn2, we3, be3,
              cbt, cbsq, e_ref, idx_ref):
    h = jnp.dot(x_ref[...].astype(BF16), we1[...],
                preferred_element_type=F32) + be1[...]
    h = _ln_elu(h, g1[...], bn1[...])
    h = jnp.dot(h.astype(BF16), we2[...],
                preferred_element_type=F32) + be2[...]
    h = _ln_elu(h, g2[...], bn2[...])
    e = jnp.dot(h.astype(BF16), we3[...],
                preferred_element_type=F32) + be3[...]
    e_ref[...] = e
    ebf2 = (e * -2.0).astype(BF16)
    esq = jnp.sum(e * e, axis=-1, keepdims=True)
    bv = jnp.full((RB, 128), jnp.inf, F32)
    bi = jnp.zeros((RB, 128), jnp.int32)
    for j in range(K // CB):
        d2 = jnp.dot(ebf2, cbt[:, j * CB:(j + 1) * CB],
                     preferred_element_type=F32)
        t = (esq + d2) + cbsq[:, j * CB:(j + 1) * CB]
        for k in range(CB // 128):
            step = j * (CB // 128) + k
            tk = lax.slice(t, (0, k * 128), (RB, (k + 1) * 128))
            pred = tk < bv
            bv = jnp.where(pred, tk, bv)
            bi = jnp.where(pred, jnp.int32(step), bi)
    lanes = lax.broadcasted_iota(jnp.int32, (RB, 128), 1)
    gidx = bi * 128 + lanes
    m = jnp.min(bv, axis=-1, keepdims=True)
    cand = jnp.where(bv == m, gidx, jnp.int32(2**31 - 1))
    idx_ref[...] = jnp.min(cand, axis=-1, keepdims=True)


def _dec_body(e_ref, q_ref, wd1, bd1, gd1, bdn1, wd2, bd2, gd2, bdn2,
              wd3, bd3, xhat_ref, loss_ref):
    hb = RB // 2
    parts = []
    for s in range(2):
        sl = pl.ds(s * hb, hb)
        e = e_ref[sl, :]
        q = q_ref[sl, :]
        diff = e - q
        parts.append(jnp.sum(jnp.sum(diff * diff, axis=-1, keepdims=True),
                             axis=0, keepdims=True))
        q_st = e + (q - e)
        h = jnp.dot(q_st.astype(BF16), wd1[...],
                    preferred_element_type=F32) + bd1[...]
        h = _ln_elu(h, gd1[...], bdn1[...])
        h = jnp.dot(h.astype(BF16), wd2[...],
                    preferred_element_type=F32) + bd2[...]
        h = _ln_elu(h, gd2[...], bdn2[...])
        xhat_ref[sl, :] = (jnp.dot(h.astype(BF16), wd3[...],
                                   preferred_element_type=F32) + bd3[...])
    part = parts[0] + parts[1]

    @pl.when(pl.program_id(0) == 0)
    def _():
        loss_ref[...] = part

    @pl.when(pl.program_id(0) != 0)
    def _():
        loss_ref[...] += part


def _full(shape):
    return pl.BlockSpec(shape, lambda i: (0,) * len(shape))


def _encode(x_bf, we1t, be1, g1, bn1, we2t, be2, g2, bn2, we3t, be3,
            cbt, cbsq):
    nb = x_bf.shape[0]
    return pl.pallas_call(
        _enc_body,
        grid=(nb // RB,),
        in_specs=[
            pl.BlockSpec((RB, D_IN), lambda i: (i, 0)),
            _full((D_IN, H1)), _full((1, H1)), _full((1, H1)), _full((1, H1)),
            _full((H1, H2)), _full((1, H2)), _full((1, H2)), _full((1, H2)),
            _full((H2, D)), _full((1, D)),
            _full((D, K)), _full((1, K)),
        ],
        out_specs=[
            pl.BlockSpec((RB, D), lambda i: (i, 0)),
            pl.BlockSpec((RB, 1), lambda i: (i, 0)),
        ],
        out_shape=[
            jax.ShapeDtypeStruct((nb, D), F32),
            jax.ShapeDtypeStruct((nb, 1), jnp.int32),
        ],
    )(x_bf, we1t, be1, g1, bn1, we2t, be2, g2, bn2, we3t, be3, cbt, cbsq)


def _decode(e, q, wd1t, bd1, gd1, bdn1, wd2t, bd2, gd2, bdn2, wd3t, bd3):
    nb = e.shape[0]
    return pl.pallas_call(
        _dec_body,
        grid=(nb // RB,),
        in_specs=[
            pl.BlockSpec((RB, D), lambda i: (i, 0)),
            pl.BlockSpec((RB, D), lambda i: (i, 0)),
            _full((D, H2)), _full((1, H2)), _full((1, H2)), _full((1, H2)),
            _full((H2, H1)), _full((1, H1)), _full((1, H1)), _full((1, H1)),
            _full((H1, D_IN)), _full((1, D_IN)),
        ],
        out_specs=[
            pl.BlockSpec((RB, D_IN), lambda i: (i, 0)),
            pl.BlockSpec((1, 1), lambda i: (0, 0)),
        ],
        out_shape=[
            jax.ShapeDtypeStruct((nb, D_IN), F32),
            jax.ShapeDtypeStruct((1, 1), F32),
        ],
    )(e, q, wd1t, bd1, gd1, bdn1, wd2t, bd2, gd2, bdn2, wd3t, bd3)


def _gather(codebook, idx):
    nb = idx.shape[0]
    info = plsc.get_sparse_core_info()
    nc, ns = info.num_cores, info.num_subcores
    nw = nc * ns
    bpw = nb // nw
    mesh = plsc.VectorSubcoreMesh(core_axis_name="c", subcore_axis_name="s")

    @functools.partial(
        pl.kernel, mesh=mesh,
        out_type=jax.ShapeDtypeStruct((nb, D), F32),
        scratch_types=[
            pltpu.VMEM((bpw,), jnp.int32),
            pltpu.VMEM((bpw, D), F32),
            pltpu.SemaphoreType.DMA,
            pltpu.SemaphoreType.DMA,
        ],
    )
    def k(table_hbm, idx_hbm, out_hbm, idx_v, rows_v, gsem, ssem):
        wid = lax.axis_index("s") * nc + lax.axis_index("c")
        base = wid * bpw
        nch = 8
        ch = bpw // nch
        pltpu.sync_copy(idx_hbm.at[pl.ds(base, bpw)], idx_v)
        gets = [pltpu.async_copy(table_hbm.at[idx_v.at[pl.ds(c * ch, ch)]],
                                 rows_v.at[pl.ds(c * ch, ch)], gsem)
                for c in range(nch)]
        puts = []
        for c in range(nch):
            gets[c].wait()
            puts.append(pltpu.async_copy(
                rows_v.at[pl.ds(c * ch, ch)],
                out_hbm.at[pl.ds(base + c * ch, ch)], ssem))
        for p in puts:
            p.wait()

    return k(codebook, idx)


def kernel(x, We1, be1, g1, bn1, We2, be2, g2, bn2, We3, be3, codebook,
           Wd1, bd1, gd1, bdn1, Wd2, bd2, gd2, bdn2, Wd3, bd3):
    we1t = We1.T.astype(BF16)
    we2t = We2.T.astype(BF16)
    we3t = We3.T.astype(BF16)
    wd1t = Wd1.T.astype(BF16)
    wd2t = Wd2.T.astype(BF16)
    wd3t = Wd3.T.astype(BF16)
    cbt = codebook.T.astype(BF16)
    cbsq = jnp.sum(codebook ** 2, axis=1).reshape(1, K)
    r = lambda v: v.reshape(1, -1)

    e, idx2 = _encode(x, we1t, r(be1), r(g1), r(bn1), we2t, r(be2),
                      r(g2), r(bn2), we3t, r(be3), cbt, cbsq)
    idx = idx2.reshape(B)
    q = _gather(codebook, idx)
    xhat, loss11 = _decode(e, q, wd1t, r(bd1), r(gd1), r(bdn1), wd2t,
                           r(bd2), r(gd2), r(bdn2), wd3t, r(bd3))
    m = loss11.reshape(()) / (B * D)
    vq_loss = m + 0.25 * m
    return (xhat, vq_loss, idx)


# speedup vs baseline: 1.0314x; 1.0026x over previous
import functools

import jax
import jax.numpy as jnp
from jax import lax
from jax.experimental import pallas as pl
from jax.experimental.pallas import tpu as pltpu
from jax.experimental.pallas import tpu_sc as plsc

F32 = jnp.float32
BF16 = jnp.bfloat16

B = 8192
D_IN = 768
H1 = 2048
H2 = 1024
D = 256
K = 8192
RB = 1024
CB = 1024
EPS = 1e-5


def _ln_elu(a, g, b):
    mu = jnp.mean(a, axis=-1, keepdims=True)
    var = jnp.mean((a - mu) ** 2, axis=-1, keepdims=True)
    y = (a - mu) / jnp.sqrt(var + EPS) * g + b
    return jnp.where(y > 0, y, jnp.exp(y) - 1.0)


def _enc_body(x_ref, we1, be1, g1, bn1, we2, be2, g2, bn2, we3, be3,
              cbt, cbsq, e_ref, idx_ref):
    h = jnp.dot(x_ref[...].astype(BF16), we1[...],
                preferred_element_type=F32) + be1[...]
    h = _ln_elu(h, g1[...], bn1[...])
    h = jnp.dot(h.astype(BF16), we2[...],
                preferred_element_type=F32) + be2[...]
    h = _ln_elu(h, g2[...], bn2[...])
    e = jnp.dot(h.astype(BF16), we3[...],
                preferred_element_type=F32) + be3[...]
    e_ref[...] = e
    ebf2 = (e * -2.0).astype(BF16)
    esq = jnp.sum(e * e, axis=-1, keepdims=True)
    bv = jnp.full((RB, 128), jnp.inf, F32)
    bi = jnp.zeros((RB, 128), jnp.int32)
    for j in range(K // CB):
        d2 = jnp.dot(ebf2, cbt[:, j * CB:(j + 1) * CB],
                     preferred_element_type=F32)
        t = (esq + d2) + cbsq[:, j * CB:(j + 1) * CB]
        for k in range(CB // 128):
            step = j * (CB // 128) + k
            tk = lax.slice(t, (0, k * 128), (RB, (k + 1) * 128))
            pred = tk < bv
            bv = jnp.where(pred, tk, bv)
            bi = jnp.where(pred, jnp.int32(step), bi)
    lanes = lax.broadcasted_iota(jnp.int32, (RB, 128), 1)
    gidx = bi * 128 + lanes
    m = jnp.min(bv, axis=-1, keepdims=True)
    cand = jnp.where(bv == m, gidx, jnp.int32(2**31 - 1))
    idx_ref[...] = jnp.min(cand, axis=-1, keepdims=True)


def _dec_body(e_ref, q_ref, wd1, bd1, gd1, bdn1, wd2, bd2, gd2, bdn2,
              wd3, bd3, xhat_ref, loss_ref):
    hb = RB // 2
    parts = []
    for s in range(2):
        sl = pl.ds(s * hb, hb)
        e = e_ref[sl, :]
        q = q_ref[sl, :]
        diff = e - q
        parts.append(jnp.sum(jnp.sum(diff * diff, axis=-1, keepdims=True),
                             axis=0, keepdims=True))
        q_st = e + (q - e)
        h = jnp.dot(q_st.astype(BF16), wd1[...],
                    preferred_element_type=F32) + bd1[...]
        h = _ln_elu(h, gd1[...], bdn1[...])
        h = jnp.dot(h.astype(BF16), wd2[...],
                    preferred_element_type=F32) + bd2[...]
        h = _ln_elu(h, gd2[...], bdn2[...])
        xhat_ref[sl, :] = (jnp.dot(h.astype(BF16), wd3[...],
                                   preferred_element_type=F32) + bd3[...])
    part = parts[0] + parts[1]

    @pl.when(pl.program_id(0) == 0)
    def _():
        loss_ref[...] = part

    @pl.when(pl.program_id(0) != 0)
    def _():
        loss_ref[...] += part


def _full(shape):
    return pl.BlockSpec(shape, lambda i: (0,) * len(shape))


def _encode(x_bf, we1t, be1, g1, bn1, we2t, be2, g2, bn2, we3t, be3,
            cbt, cbsq):
    nb = x_bf.shape[0]
    return pl.pallas_call(
        _enc_body,
        grid=(nb // RB,),
        in_specs=[
            pl.BlockSpec((RB, D_IN), lambda i: (i, 0)),
            _full((D_IN, H1)), _full((1, H1)), _full((1, H1)), _full((1, H1)),
            _full((H1, H2)), _full((1, H2)), _full((1, H2)), _full((1, H2)),
            _full((H2, D)), _full((1, D)),
            _full((D, K)), _full((1, K)),
        ],
        out_specs=[
            pl.BlockSpec((RB, D), lambda i: (i, 0)),
            pl.BlockSpec((RB, 1), lambda i: (i, 0)),
        ],
        out_shape=[
            jax.ShapeDtypeStruct((nb, D), F32),
            jax.ShapeDtypeStruct((nb, 1), jnp.int32),
        ],
    )(x_bf, we1t, be1, g1, bn1, we2t, be2, g2, bn2, we3t, be3, cbt, cbsq)


def _decode(e, q, wd1t, bd1, gd1, bdn1, wd2t, bd2, gd2, bdn2, wd3t, bd3):
    nb = e.shape[0]
    return pl.pallas_call(
        _dec_body,
        grid=(nb // RB,),
        in_specs=[
            pl.BlockSpec((RB, D), lambda i: (i, 0)),
            pl.BlockSpec((RB, D), lambda i: (i, 0)),
            _full((D, H2)), _full((1, H2)), _full((1, H2)), _full((1, H2)),
            _full((H2, H1)), _full((1, H1)), _full((1, H1)), _full((1, H1)),
            _full((H1, D_IN)), _full((1, D_IN)),
        ],
        out_specs=[
            pl.BlockSpec((RB, D_IN), lambda i: (i, 0)),
            pl.BlockSpec((1, 1), lambda i: (0, 0)),
        ],
        out_shape=[
            jax.ShapeDtypeStruct((nb, D_IN), F32),
            jax.ShapeDtypeStruct((1, 1), F32),
        ],
    )(e, q, wd1t, bd1, gd1, bdn1, wd2t, bd2, gd2, bdn2, wd3t, bd3)


def _gather(codebook, idx):
    nb = idx.shape[0]
    info = plsc.get_sparse_core_info()
    nc, ns = info.num_cores, info.num_subcores
    nw = nc * ns
    bpw = nb // nw
    mesh = plsc.VectorSubcoreMesh(core_axis_name="c", subcore_axis_name="s")

    @functools.partial(
        pl.kernel, mesh=mesh,
        out_type=jax.ShapeDtypeStruct((nb, D), F32),
        scratch_types=[
            pltpu.VMEM((bpw,), jnp.int32),
            pltpu.VMEM((bpw, D), F32),
            pltpu.SemaphoreType.DMA,
            pltpu.SemaphoreType.DMA,
        ],
    )
    def k(table_hbm, idx_hbm, out_hbm, idx_v, rows_v, gsem, ssem):
        wid = lax.axis_index("s") * nc + lax.axis_index("c")
        base = wid * bpw
        nch = 8
        ch = bpw // nch
        pltpu.sync_copy(idx_hbm.at[pl.ds(base, bpw)], idx_v)
        gets = [pltpu.async_copy(table_hbm.at[idx_v.at[pl.ds(c * ch, ch)]],
                                 rows_v.at[pl.ds(c * ch, ch)], gsem)
                for c in range(nch)]
        puts = []
        for c in range(nch):
            gets[c].wait()
            puts.append(pltpu.async_copy(
                rows_v.at[pl.ds(c * ch, ch)],
                out_hbm.at[pl.ds(base + c * ch, ch)], ssem))
        for p in puts:
            p.wait()

    return k(codebook, idx)


def kernel(x, We1, be1, g1, bn1, We2, be2, g2, bn2, We3, be3, codebook,
           Wd1, bd1, gd1, bdn1, Wd2, bd2, gd2, bdn2, Wd3, bd3):
    we1t = We1.T.astype(BF16)
    we2t = We2.T.astype(BF16)
    we3t = We3.T.astype(BF16)
    wd1t = Wd1.T.astype(BF16)
    wd2t = Wd2.T.astype(BF16)
    wd3t = Wd3.T.astype(BF16)
    cbt = codebook.T.astype(BF16)
    cbsq = jnp.sum(codebook ** 2, axis=1).reshape(1, K)
    r = lambda v: v.reshape(1, -1)

    e, idx2 = _encode(x, we1t, r(be1), r(g1), r(bn1), we2t, r(be2),
                      r(g2), r(bn2), we3t, r(be3), cbt, cbsq)
    idx = idx2.reshape(B)
    q = _gather(codebook, idx)
    xhat, loss11 = _decode(e, q, wd1t, r(bd1), r(gd1), r(bdn1), wd2t,
                           r(bd2), r(gd2), r(bdn2), wd3t, r(bd3))
    m = loss11.reshape(()) / (B * D)
    vq_loss = m + 0.25 * m
    return (xhat, vq_loss, idx)
